# baseline (speedup 1.0000x reference)
import functools

import jax
import jax.numpy as jnp
from jax import lax
from jax.experimental import pallas as pl
from jax.experimental.pallas import tpu as pltpu
from jax.experimental.pallas import tpu_sc as plsc

VOCAB = 1000
EMBED_DIM = 32
BATCH = 16384
FIELDS = 100

_INFO = plsc.get_sparse_core_info()
_NC = _INFO.num_cores
_NS = _INFO.num_subcores
_NW = _NC * _NS
_COLS = BATCH // _NW
_NVEC = _COLS // 16


def _emb_body(tt_hbm, xt_hbm, out_hbm,
              tab_v, idx_v, out_A, out_B,
              sem_i0, sem_i1, sem_i2, sem_i3, sem_oA, sem_oB):
    wid = lax.axis_index("s") * _NC + lax.axis_index("c")
    col0 = wid * _COLS

    pltpu.sync_copy(tt_hbm, tab_v)

    dvecs = [jnp.full((16,), d, jnp.int32) for d in range(EMBED_DIM)]
    isems = [sem_i0, sem_i1, sem_i2, sem_i3]

    def compute(slot, out_v):
        @plsc.parallel_loop(0, _NVEC, 1, unroll=2)
        def group(i):
            idx16 = idx_v[slot, pl.ds(i * 16, 16)]
            vals = [
                plsc.load_gather(tab_v, [dvecs[d], idx16])
                for d in range(EMBED_DIM)
            ]
            for d in range(EMBED_DIM):
                out_v[d, pl.ds(i * 16, 16)] = vals[d]

    def idx_start(f, slot):
        pltpu.async_copy(
            xt_hbm.at[f, pl.ds(col0, _COLS)], idx_v.at[slot], isems[slot])

    def idx_wait(f, slot):
        pltpu.make_async_copy(
            xt_hbm.at[f, pl.ds(col0, _COLS)], idx_v.at[slot], isems[slot]).wait()

    def out_start(f, buf, sem):
        pltpu.async_copy(buf, out_hbm.at[pl.ds(f, 2), :, pl.ds(col0, _COLS)], sem)

    def out_wait(f, buf, sem):
        pltpu.make_async_copy(
            buf, out_hbm.at[pl.ds(f, 2), :, pl.ds(col0, _COLS)], sem).wait()

    idx_start(0, 0)
    idx_start(1, 1)

    def quad(g, carry):
        f0 = 4 * g
        idx_start(f0 + 2, 2)
        idx_start(f0 + 3, 3)

        idx_wait(f0, 0)

        @pl.when(g > 0)
        def _():
            out_wait(f0, out_A, sem_oA)

        compute(0, out_A.at[0])
        idx_wait(f0 + 1, 1)
        compute(1, out_A.at[1])
        out_start(f0, out_A, sem_oA)

        @pl.when(f0 + 4 < FIELDS)
        def _():
            idx_start(f0 + 4, 0)
            idx_start(f0 + 5, 1)

        idx_wait(f0 + 2, 2)

        @pl.when(g > 0)
        def _():
            out_wait(f0 + 2, out_B, sem_oB)

        compute(2, out_B.at[0])
        idx_wait(f0 + 3, 3)
        compute(3, out_B.at[1])
        out_start(f0 + 2, out_B, sem_oB)
        return carry

    lax.fori_loop(0, FIELDS // 4, quad, 0)
    out_wait(FIELDS - 4, out_A, sem_oA)
    out_wait(FIELDS - 2, out_B, sem_oB)


_emb = pl.kernel(
    _emb_body,
    out_type=jax.ShapeDtypeStruct((FIELDS, EMBED_DIM, BATCH), jnp.float32),
    mesh=plsc.VectorSubcoreMesh(core_axis_name="c", subcore_axis_name="s"),
    compiler_params=pltpu.CompilerParams(needs_layout_passes=False),
    scratch_types=[
        pltpu.VMEM((EMBED_DIM, VOCAB), jnp.float32),
        pltpu.VMEM((4, _COLS), jnp.int32),
        pltpu.VMEM((2, EMBED_DIM, _COLS), jnp.float32),
        pltpu.VMEM((2, EMBED_DIM, _COLS), jnp.float32),
        pltpu.SemaphoreType.DMA,
        pltpu.SemaphoreType.DMA,
        pltpu.SemaphoreType.DMA,
        pltpu.SemaphoreType.DMA,
        pltpu.SemaphoreType.DMA,
        pltpu.SemaphoreType.DMA,
    ],
)


def kernel(x, table):
    xt = x.T.astype(jnp.int32)
    tt = table.T
    outt = _emb(tt, xt)
    return outt.transpose(2, 0, 1)


# speedup vs baseline: 1.4251x; 1.4251x over previous
import functools

import jax
import jax.numpy as jnp
from jax import lax
from jax.experimental import pallas as pl
from jax.experimental.pallas import tpu as pltpu
from jax.experimental.pallas import tpu_sc as plsc

VOCAB = 1000
EMBED_DIM = 32
BATCH = 16384
FIELDS = 100

_INFO = plsc.get_sparse_core_info()
_NC = _INFO.num_cores
_NS = _INFO.num_subcores
_NW = _NC * _NS
_COLS = BATCH // _NW
_NVEC = _COLS // 16


_NBUF = 4


def _emb_body(tt_hbm, xt_hbm, out_hbm,
              tab_v, idx_bufs, out_bufs, isems, osems):
    wid = lax.axis_index("s") * _NC + lax.axis_index("c")
    col0 = wid * _COLS

    pltpu.sync_copy(tt_hbm, tab_v)

    dvecs = [jnp.full((16,), d, jnp.int32) for d in range(EMBED_DIM)]

    def compute(idx_v, out_v):
        @plsc.parallel_loop(0, _NVEC, 1, unroll=2)
        def group(i):
            idx16 = idx_v[pl.ds(i * 16, 16)]
            vals = [
                plsc.load_gather(tab_v, [dvecs[d], idx16])
                for d in range(EMBED_DIM)
            ]
            for d in range(EMBED_DIM):
                out_v[d, pl.ds(i * 16, 16)] = vals[d]

    def idx_start(f, k):
        pltpu.async_copy(xt_hbm.at[f, pl.ds(col0, _COLS)], idx_bufs[k], isems[k])

    def idx_wait(f, k):
        pltpu.make_async_copy(
            xt_hbm.at[f, pl.ds(col0, _COLS)], idx_bufs[k], isems[k]).wait()

    def out_start(f, k):
        pltpu.async_copy(
            out_bufs[k], out_hbm.at[f, :, pl.ds(col0, _COLS)], osems[k])

    def out_wait(f, k):
        pltpu.make_async_copy(
            out_bufs[k], out_hbm.at[f, :, pl.ds(col0, _COLS)], osems[k]).wait()

    for k in range(_NBUF):
        idx_start(k, k)

    def quad(g, carry):
        f0 = _NBUF * g
        for k in range(_NBUF):
            f = f0 + k
            idx_wait(f, k)

            @pl.when(g > 0)
            def _():
                out_wait(f - _NBUF, k)

            compute(idx_bufs[k], out_bufs[k])
            out_start(f, k)

            @pl.when(f + _NBUF < FIELDS)
            def _():
                idx_start(f + _NBUF, k)

        return carry

    lax.fori_loop(0, FIELDS // _NBUF, quad, 0)
    for k in range(_NBUF):
        out_wait(FIELDS - _NBUF + k, k)


_emb = pl.kernel(
    _emb_body,
    out_type=jax.ShapeDtypeStruct((FIELDS, EMBED_DIM, BATCH), jnp.float32),
    mesh=plsc.VectorSubcoreMesh(core_axis_name="c", subcore_axis_name="s"),
    compiler_params=pltpu.CompilerParams(needs_layout_passes=False),
    scratch_types=[
        pltpu.VMEM((EMBED_DIM, VOCAB), jnp.float32),
        [pltpu.VMEM((_COLS,), jnp.int32) for _ in range(_NBUF)],
        [pltpu.VMEM((EMBED_DIM, _COLS), jnp.float32) for _ in range(_NBUF)],
        [pltpu.SemaphoreType.DMA for _ in range(_NBUF)],
        [pltpu.SemaphoreType.DMA for _ in range(_NBUF)],
    ],
)


def kernel(x, table):
    xt = x.T.astype(jnp.int32)
    tt = table.T
    outt = _emb(tt, xt)
    return outt.transpose(2, 0, 1)
